# Initial kernel scaffold; baseline (speedup 1.0000x reference)
import jax
import jax.numpy as jnp
from jax.experimental import pallas as pl


def kernel(x, edge_index, edge_attr, W1, b1, W2, b2):
    raise NotImplementedError("write your pallas kernel here")


# speedup vs baseline: 2.1356x; 2.1356x over previous
import functools

import jax
import jax.numpy as jnp
from jax import lax
from jax.experimental import pallas as pl
from jax.experimental.pallas import tpu as pltpu
from jax.experimental.pallas import tpu_sc as plsc

_N_NODES = 10000
_N_EDGES = 320000
_D = 128


_EB = 3200


def _payload_body(ea_ref, w1t_ref, b1_ref, out_ref):
    a = ea_ref[...]
    h = jax.lax.dot_general(
        a, w1t_ref[...], (((1,), (0,)), ((), ())),
        preferred_element_type=jnp.float32,
        precision=jax.lax.Precision.HIGHEST)
    h = jnp.maximum(h + b1_ref[...], 0.0)
    w = 1.0 / (1.0 + jnp.exp(-a[:, 2:3]))
    out_ref[...] = h * w


def _payload(edge_attr, W1, b1):
    grid = (_N_EDGES // _EB,)
    return pl.pallas_call(
        _payload_body,
        grid=grid,
        in_specs=[
            pl.BlockSpec((_EB, 4), lambda i: (i, 0)),
            pl.BlockSpec((4, _D), lambda i: (0, 0)),
            pl.BlockSpec((1, _D), lambda i: (0, 0)),
        ],
        out_specs=pl.BlockSpec((_EB, _D), lambda i: (i, 0)),
        out_shape=jax.ShapeDtypeStruct((_N_EDGES, _D), jnp.float32),
    )(edge_attr, W1.T, b1[None, :])


_NC, _NS = 2, 16
_NW = _NC * _NS
_EPT = _N_EDGES // _NW
_CH = 128
_NCHUNK = _EPT // _CH
_TAIL = _EPT - _NCHUNK * _CH
_RPT = 624
_RREM = _N_NODES - _NS * _RPT
_L = 16


def _sigmoid16(v):
    return 1.0 / (1.0 + jnp.exp(-v))


def _sc_body(g_hbm, dst_hbm, ea2_hbm, zeros_hbm, out_hbm, outs_hbm,
             rows_v, idx_v, a2_v, rows_t, idx_t, a2_t, s_v, acc_sh):
    c = lax.axis_index("c")
    s = lax.axis_index("s")
    wid = s * _NC + c
    r0 = s * _RPT
    rr = _NS * _RPT
    pltpu.sync_copy(zeros_hbm.at[pl.ds(r0, _RPT)], acc_sh.at[pl.ds(r0, _RPT)])

    @pl.when(s == _NS - 1)
    def _():
        pltpu.sync_copy(zeros_hbm.at[pl.ds(rr, _RREM)],
                        acc_sh.at[pl.ds(rr, _RREM)])

    def zbody(i, carry):
        s_v[pl.ds(i * _L, _L)] = jnp.zeros((_L,), jnp.float32)
        return carry

    lax.fori_loop(0, _N_NODES // _L, zbody, 0)
    plsc.subcore_barrier()

    base = wid * _EPT

    def body(j, carry):
        off = pl.multiple_of(base + j * _CH, 8)
        pltpu.sync_copy(dst_hbm.at[pl.ds(off, _CH)], idx_v)
        pltpu.sync_copy(g_hbm.at[pl.ds(off, _CH)], rows_v)
        pltpu.sync_copy(ea2_hbm.at[pl.ds(off, _CH)], a2_v)
        pltpu.sync_copy(rows_v, acc_sh.at[idx_v], add=True)
        for k in range(_CH // _L):
            idx16 = idx_v[pl.ds(k * _L, _L)]
            wv = _sigmoid16(a2_v[pl.ds(k * _L, _L)])
            plsc.addupdate_scatter(s_v, [idx16], wv)
        return carry

    lax.fori_loop(0, _NCHUNK, body, 0)
    offt = base + _NCHUNK * _CH
    pltpu.sync_copy(dst_hbm.at[pl.ds(offt, _TAIL)], idx_t)
    pltpu.sync_copy(g_hbm.at[pl.ds(offt, _TAIL)], rows_t)
    pltpu.sync_copy(ea2_hbm.at[pl.ds(offt, _TAIL)], a2_t)
    pltpu.sync_copy(rows_t, acc_sh.at[idx_t], add=True)
    for k in range(_TAIL // _L):
        idx16 = idx_t[pl.ds(k * _L, _L)]
        wv = _sigmoid16(a2_t[pl.ds(k * _L, _L)])
        plsc.addupdate_scatter(s_v, [idx16], wv)

    plsc.subcore_barrier()
    pltpu.sync_copy(acc_sh.at[pl.ds(r0, _RPT)], out_hbm.at[c, pl.ds(r0, _RPT)])

    @pl.when(s == _NS - 1)
    def _():
        pltpu.sync_copy(acc_sh.at[pl.ds(rr, _RREM)],
                        out_hbm.at[c, pl.ds(rr, _RREM)])

    pltpu.sync_copy(s_v, outs_hbm.at[wid])


def _sc_scatter(payload, dst, ea2, zeros):
    mesh = plsc.VectorSubcoreMesh(core_axis_name="c", subcore_axis_name="s")
    fn = functools.partial(
        pl.kernel,
        mesh=mesh,
        compiler_params=pltpu.CompilerParams(needs_layout_passes=False),
        out_type=[
            jax.ShapeDtypeStruct((_NC, _N_NODES, _D), jnp.float32),
            jax.ShapeDtypeStruct((_NW, _N_NODES), jnp.float32),
        ],
        scratch_types=[
            pltpu.VMEM((_CH, _D), jnp.float32),
            pltpu.VMEM((_CH,), jnp.int32),
            pltpu.VMEM((_CH,), jnp.float32),
            pltpu.VMEM((_TAIL, _D), jnp.float32),
            pltpu.VMEM((_TAIL,), jnp.int32),
            pltpu.VMEM((_TAIL,), jnp.float32),
            pltpu.VMEM((_N_NODES,), jnp.float32),
            pltpu.VMEM_SHARED((_N_NODES, _D), jnp.float32),
        ],
    )(_sc_body)
    return fn(payload, dst, ea2, zeros)


_NB = 1000


def _final_body(x_ref, acc_ref, s_ref, w2_ref, b2_ref, out_ref):
    a = acc_ref[0] + acc_ref[1]
    y = jax.lax.dot_general(
        a, w2_ref[...], (((1,), (1,)), ((), ())),
        preferred_element_type=jnp.float32,
        precision=jax.lax.Precision.HIGHEST)
    sw = jnp.sum(s_ref[...], axis=1, keepdims=True)
    out_ref[...] = x_ref[...] + y + sw * b2_ref[...]


def _final(x, acc, s_t, W2, b2):
    grid = (_N_NODES // _NB,)
    return pl.pallas_call(
        _final_body,
        grid=grid,
        in_specs=[
            pl.BlockSpec((_NB, _D), lambda i: (i, 0)),
            pl.BlockSpec((_NC, _NB, _D), lambda i: (0, i, 0)),
            pl.BlockSpec((_NB, _NW), lambda i: (i, 0)),
            pl.BlockSpec((_D, _D), lambda i: (0, 0)),
            pl.BlockSpec((1, _D), lambda i: (0, 0)),
        ],
        out_specs=pl.BlockSpec((_NB, _D), lambda i: (i, 0)),
        out_shape=jax.ShapeDtypeStruct((_N_NODES, _D), jnp.float32),
    )(x, acc, s_t, W2, b2[None, :])


def kernel(x, edge_index, edge_attr, W1, b1, W2, b2):
    dst = edge_index[1].astype(jnp.int32)
    ea2 = edge_attr[:, 2]
    payload = _payload(edge_attr, W1, b1)
    zeros = jnp.zeros((_N_NODES, _D), jnp.float32)
    acc, s_parts = _sc_scatter(payload, dst, ea2, zeros)
    return _final(x, acc, s_parts.T, W2, b2)


# speedup vs baseline: 2.7138x; 1.2707x over previous
import functools

import jax
import jax.numpy as jnp
from jax import lax
from jax.experimental import pallas as pl
from jax.experimental.pallas import tpu as pltpu
from jax.experimental.pallas import tpu_sc as plsc

_N_NODES = 10000
_N_EDGES = 320000
_D = 128

_NC, _NS = 2, 16
_NW = _NC * _NS
_CH = 128
_CPT = 80
_EPT = _CPT * _CH
_EPAD = _NW * _EPT
_NACC = _N_NODES + 16
_RPT = 624
_L = 16


_EB = 2560
_LAST_FULL = _N_EDGES // _EB - 1


def _payload_body(ea_ref, w1t_ref, b1_ref, out_ref):
    a = ea_ref[...]
    h = jax.lax.dot_general(
        a, w1t_ref[...], (((1,), (0,)), ((), ())),
        preferred_element_type=jnp.float32,
        precision=jax.lax.Precision.HIGHEST)
    h = jnp.maximum(h + b1_ref[...], 0.0)
    w = 1.0 / (1.0 + jnp.exp(-a[:, 2:3]))
    out_ref[...] = h * w


def _payload(edge_attr, W1, b1):
    grid = (_EPAD // _EB,)
    return pl.pallas_call(
        _payload_body,
        grid=grid,
        in_specs=[
            pl.BlockSpec((_EB, 4), lambda i: (jnp.minimum(i, _LAST_FULL), 0)),
            pl.BlockSpec((4, _D), lambda i: (0, 0)),
            pl.BlockSpec((1, _D), lambda i: (0, 0)),
        ],
        out_specs=pl.BlockSpec((_EB, _D), lambda i: (i, 0)),
        out_shape=jax.ShapeDtypeStruct((_EPAD, _D), jnp.float32),
    )(edge_attr, W1.T, b1[None, :])


def _sigmoid16(v):
    return 1.0 / (1.0 + jnp.exp(-v))


def _sc_body(g_hbm, idx_hbm, ea2_hbm, zeros_hbm, outa_hbm, outs_hbm,
             rows0, rows1, idx0, idx1, ea0, ea1, s_v, acc_sh, sem0, sem1):
    c = lax.axis_index("c")
    s = lax.axis_index("s")
    wid = s * _NC + c
    r0 = s * _RPT
    rr = _NS * _RPT
    pltpu.sync_copy(zeros_hbm.at[pl.ds(r0, _RPT)], acc_sh.at[pl.ds(r0, _RPT)])

    @pl.when(s == _NS - 1)
    def _():
        pltpu.sync_copy(zeros_hbm.at[pl.ds(rr, _NACC - rr)],
                        acc_sh.at[pl.ds(rr, _NACC - rr)])

    def zbody(i, carry):
        s_v[pl.ds(i * _L, _L)] = jnp.zeros((_L,), jnp.float32)
        return carry

    lax.fori_loop(0, _NACC // _L, zbody, 0)

    ebase = wid * _EPT
    bufs = (rows0, rows1)
    idxs = (idx0, idx1)
    eas = (ea0, ea1)
    sems = (sem0, sem1)

    def start(off, slot):
        pltpu.async_copy(g_hbm.at[pl.ds(off, _CH)], bufs[slot], sems[slot])
        pltpu.async_copy(idx_hbm.at[pl.ds(off, _CH)], idxs[slot], sems[slot])
        pltpu.async_copy(ea2_hbm.at[pl.ds(off, _CH)], eas[slot], sems[slot])

    start(ebase, 0)

    def outer(gi, carry):
        for b in range(2):
            j = gi * 2 + b

            @pl.when(j + 1 < _CPT)
            def _():
                start(ebase + (j + 1) * _CH, 1 - b)

            pltpu.make_async_copy(
                g_hbm.at[pl.ds(0, _CH)], bufs[b], sems[b]).wait()
            pltpu.make_async_copy(
                idx_hbm.at[pl.ds(0, _CH)], idxs[b], sems[b]).wait()
            pltpu.make_async_copy(
                ea2_hbm.at[pl.ds(0, _CH)], eas[b], sems[b]).wait()
            pltpu.sync_copy(bufs[b], acc_sh.at[idxs[b]], add=True)
            for k in range(_CH // _L):
                idx16 = idxs[b][pl.ds(k * _L, _L)]
                wv = _sigmoid16(eas[b][pl.ds(k * _L, _L)])
                plsc.addupdate_scatter(s_v, [idx16], wv)
        return carry

    lax.fori_loop(0, _CPT // 2, outer, 0)

    plsc.subcore_barrier()
    pltpu.sync_copy(acc_sh.at[pl.ds(r0, _RPT)],
                    outa_hbm.at[c, pl.ds(r0, _RPT)])

    @pl.when(s == _NS - 1)
    def _():
        pltpu.sync_copy(acc_sh.at[pl.ds(rr, _N_NODES - rr)],
                        outa_hbm.at[c, pl.ds(rr, _N_NODES - rr)])

    pltpu.sync_copy(s_v, outs_hbm.at[wid])


def _sc_scatter(payload, idx1d, ea2_1d, zeros):
    mesh = plsc.VectorSubcoreMesh(core_axis_name="c", subcore_axis_name="s")
    fn = functools.partial(
        pl.kernel,
        mesh=mesh,
        compiler_params=pltpu.CompilerParams(needs_layout_passes=False),
        out_type=[
            jax.ShapeDtypeStruct((_NC, _N_NODES, _D), jnp.float32),
            jax.ShapeDtypeStruct((_NW, _NACC), jnp.float32),
        ],
        scratch_types=[
            pltpu.VMEM((_CH, _D), jnp.float32),
            pltpu.VMEM((_CH, _D), jnp.float32),
            pltpu.VMEM((_CH,), jnp.int32),
            pltpu.VMEM((_CH,), jnp.int32),
            pltpu.VMEM((_CH,), jnp.float32),
            pltpu.VMEM((_CH,), jnp.float32),
            pltpu.VMEM((_NACC,), jnp.float32),
            pltpu.VMEM_SHARED((_NACC, _D), jnp.float32),
            pltpu.SemaphoreType.DMA,
            pltpu.SemaphoreType.DMA,
        ],
    )(_sc_body)
    return fn(payload, idx1d, ea2_1d, zeros)


_NB = 1000


def _final_body(x_ref, acc_ref, s_ref, w2_ref, b2_ref, out_ref):
    a = acc_ref[0] + acc_ref[1]
    y = jax.lax.dot_general(
        a, w2_ref[...], (((1,), (1,)), ((), ())),
        preferred_element_type=jnp.float32,
        precision=jax.lax.Precision.HIGHEST)
    sw = jnp.sum(s_ref[...], axis=1, keepdims=True)
    out_ref[...] = x_ref[...] + y + sw * b2_ref[...]


def _final(x, acc, s_t, W2, b2):
    grid = (_N_NODES // _NB,)
    return pl.pallas_call(
        _final_body,
        grid=grid,
        in_specs=[
            pl.BlockSpec((_NB, _D), lambda i: (i, 0)),
            pl.BlockSpec((_NC, _NB, _D), lambda i: (0, i, 0)),
            pl.BlockSpec((_NB, _NW), lambda i: (i, 0)),
            pl.BlockSpec((_D, _D), lambda i: (0, 0)),
            pl.BlockSpec((1, _D), lambda i: (0, 0)),
        ],
        out_specs=pl.BlockSpec((_NB, _D), lambda i: (i, 0)),
        out_shape=jax.ShapeDtypeStruct((_N_NODES, _D), jnp.float32),
    )(x, acc, s_t, W2, b2[None, :])


def kernel(x, edge_index, edge_attr, W1, b1, W2, b2):
    npad = _EPAD - _N_EDGES
    dst = edge_index[1].astype(jnp.int32)
    idx1d = jnp.concatenate([dst, jnp.full((npad,), _N_NODES, jnp.int32)])
    ea2_1d = jnp.concatenate([edge_attr[:, 2], jnp.zeros((npad,), jnp.float32)])
    payload = _payload(edge_attr, W1, b1)
    zeros = jnp.zeros((_NACC, _D), jnp.float32)
    acc, s_parts = _sc_scatter(payload, idx1d, ea2_1d, zeros)
    return _final(x, acc, s_parts.T, W2, b2)


# speedup vs baseline: 2.9815x; 1.0986x over previous
import functools

import jax
import jax.numpy as jnp
from jax import lax
from jax.experimental import pallas as pl
from jax.experimental.pallas import tpu as pltpu
from jax.experimental.pallas import tpu_sc as plsc

_N_NODES = 10000
_N_EDGES = 320000
_D = 128

_NC, _NS = 2, 16
_NW = _NC * _NS
_CH = 128
_CPT = 80
_EPT = _CPT * _CH
_EPAD = _NW * _EPT
_NACC = _N_NODES + 16
_RPT = 624
_L = 16


_EB = 2560
_LAST_FULL = _N_EDGES // _EB - 1


def _payload_body(ea_ref, w1t_ref, b1_ref, out_ref):
    a = ea_ref[...]
    h = jax.lax.dot_general(
        a, w1t_ref[...], (((1,), (0,)), ((), ())),
        preferred_element_type=jnp.float32,
        precision=jax.lax.Precision.DEFAULT)
    h = jnp.maximum(h + b1_ref[...], 0.0)
    w = 1.0 / (1.0 + jnp.exp(-a[:, 2:3]))
    out_ref[...] = h * w


def _payload(edge_attr, W1, b1):
    grid = (_EPAD // _EB,)
    return pl.pallas_call(
        _payload_body,
        grid=grid,
        in_specs=[
            pl.BlockSpec((_EB, 4), lambda i: (jnp.minimum(i, _LAST_FULL), 0)),
            pl.BlockSpec((4, _D), lambda i: (0, 0)),
            pl.BlockSpec((1, _D), lambda i: (0, 0)),
        ],
        out_specs=pl.BlockSpec((_EB, _D), lambda i: (i, 0)),
        out_shape=jax.ShapeDtypeStruct((_EPAD, _D), jnp.float32),
    )(edge_attr, W1.T, b1[None, :])


def _sigmoid16(v):
    return 1.0 / (1.0 + jnp.exp(-v))


def _sc_body(g_hbm, idx_hbm, ea2_hbm, zeros_hbm, outa_hbm, outs_hbm,
             rows0, rows1, idx0, idx1, ea0, ea1, s_v, acc_sh, sem0, sem1):
    c = lax.axis_index("c")
    s = lax.axis_index("s")
    wid = s * _NC + c
    r0 = s * _RPT
    rr = _NS * _RPT
    pltpu.sync_copy(zeros_hbm.at[pl.ds(r0, _RPT)], acc_sh.at[pl.ds(r0, _RPT)])

    @pl.when(s == _NS - 1)
    def _():
        pltpu.sync_copy(zeros_hbm.at[pl.ds(rr, _NACC - rr)],
                        acc_sh.at[pl.ds(rr, _NACC - rr)])

    def zbody(i, carry):
        s_v[pl.ds(i * _L, _L)] = jnp.zeros((_L,), jnp.float32)
        return carry

    lax.fori_loop(0, _NACC // _L, zbody, 0)

    ebase = wid * _EPT
    bufs = (rows0, rows1)
    idxs = (idx0, idx1)
    eas = (ea0, ea1)
    sems = (sem0, sem1)

    def start(off, slot):
        pltpu.async_copy(g_hbm.at[pl.ds(off, _CH)], bufs[slot], sems[slot])
        pltpu.async_copy(idx_hbm.at[pl.ds(off, _CH)], idxs[slot], sems[slot])
        pltpu.async_copy(ea2_hbm.at[pl.ds(off, _CH)], eas[slot], sems[slot])

    start(ebase, 0)

    def outer(gi, carry):
        for b in range(2):
            j = gi * 2 + b

            @pl.when(j + 1 < _CPT)
            def _():
                start(ebase + (j + 1) * _CH, 1 - b)

            pltpu.make_async_copy(
                g_hbm.at[pl.ds(0, _CH)], bufs[b], sems[b]).wait()
            pltpu.make_async_copy(
                idx_hbm.at[pl.ds(0, _CH)], idxs[b], sems[b]).wait()
            pltpu.make_async_copy(
                ea2_hbm.at[pl.ds(0, _CH)], eas[b], sems[b]).wait()
            pltpu.sync_copy(bufs[b], acc_sh.at[idxs[b]], add=True)
            for k in range(_CH // _L):
                idx16 = idxs[b][pl.ds(k * _L, _L)]
                wv = _sigmoid16(eas[b][pl.ds(k * _L, _L)])
                plsc.addupdate_scatter(s_v, [idx16], wv)
        return carry

    lax.fori_loop(0, _CPT // 2, outer, 0)

    plsc.subcore_barrier()
    pltpu.sync_copy(acc_sh.at[pl.ds(r0, _RPT)],
                    outa_hbm.at[c, pl.ds(r0, _RPT)])

    @pl.when(s == _NS - 1)
    def _():
        pltpu.sync_copy(acc_sh.at[pl.ds(rr, _N_NODES - rr)],
                        outa_hbm.at[c, pl.ds(rr, _N_NODES - rr)])

    pltpu.sync_copy(s_v, outs_hbm.at[wid])


def _sc_scatter(payload, idx1d, ea2_1d, zeros):
    mesh = plsc.VectorSubcoreMesh(core_axis_name="c", subcore_axis_name="s")
    fn = functools.partial(
        pl.kernel,
        mesh=mesh,
        compiler_params=pltpu.CompilerParams(needs_layout_passes=False),
        out_type=[
            jax.ShapeDtypeStruct((_NC, _N_NODES, _D), jnp.float32),
            jax.ShapeDtypeStruct((_NW, _NACC), jnp.float32),
        ],
        scratch_types=[
            pltpu.VMEM((_CH, _D), jnp.float32),
            pltpu.VMEM((_CH, _D), jnp.float32),
            pltpu.VMEM((_CH,), jnp.int32),
            pltpu.VMEM((_CH,), jnp.int32),
            pltpu.VMEM((_CH,), jnp.float32),
            pltpu.VMEM((_CH,), jnp.float32),
            pltpu.VMEM((_NACC,), jnp.float32),
            pltpu.VMEM_SHARED((_NACC, _D), jnp.float32),
            pltpu.SemaphoreType.DMA,
            pltpu.SemaphoreType.DMA,
        ],
    )(_sc_body)
    return fn(payload, idx1d, ea2_1d, zeros)


_NB = 1000


def _final_body(x_ref, acc_ref, s_ref, w2_ref, b2_ref, out_ref):
    a = acc_ref[0] + acc_ref[1]
    y = jax.lax.dot_general(
        a, w2_ref[...], (((1,), (1,)), ((), ())),
        preferred_element_type=jnp.float32,
        precision=jax.lax.Precision.HIGHEST)
    sw = jnp.sum(s_ref[...], axis=1, keepdims=True)
    out_ref[...] = x_ref[...] + y + sw * b2_ref[...]


def _final(x, acc, s_t, W2, b2):
    grid = (_N_NODES // _NB,)
    return pl.pallas_call(
        _final_body,
        grid=grid,
        in_specs=[
            pl.BlockSpec((_NB, _D), lambda i: (i, 0)),
            pl.BlockSpec((_NC, _NB, _D), lambda i: (0, i, 0)),
            pl.BlockSpec((_NB, _NW), lambda i: (i, 0)),
            pl.BlockSpec((_D, _D), lambda i: (0, 0)),
            pl.BlockSpec((1, _D), lambda i: (0, 0)),
        ],
        out_specs=pl.BlockSpec((_NB, _D), lambda i: (i, 0)),
        out_shape=jax.ShapeDtypeStruct((_N_NODES, _D), jnp.float32),
    )(x, acc, s_t, W2, b2[None, :])


def kernel(x, edge_index, edge_attr, W1, b1, W2, b2):
    npad = _EPAD - _N_EDGES
    dst = edge_index[1].astype(jnp.int32)
    pad_idx = _N_NODES + (jnp.arange(npad, dtype=jnp.int32) % 16)
    idx1d = jnp.concatenate([dst, pad_idx])
    ea2_1d = jnp.concatenate([edge_attr[:, 2], jnp.zeros((npad,), jnp.float32)])
    payload = _payload(edge_attr, W1, b1)
    zeros = jnp.zeros((_NACC, _D), jnp.float32)
    acc, s_parts = _sc_scatter(payload, idx1d, ea2_1d, zeros)
    return _final(x, acc, s_parts.T, W2, b2)


# speedup vs baseline: 4.5673x; 1.5319x over previous
import functools

import jax
import jax.numpy as jnp
from jax import lax
from jax.experimental import pallas as pl
from jax.experimental.pallas import tpu as pltpu
from jax.experimental.pallas import tpu_sc as plsc

_N_NODES = 10000
_N_EDGES = 320000
_D = 128

_NC, _NS = 2, 16
_NW = _NC * _NS
_CH = 128
_CPT = 80
_EPT = _CPT * _CH
_EPAD = _NW * _EPT
_NACC = _N_NODES + 16
_RPT = 624
_L = 16


_EB = 10240


_DW = _D + 16


def _payload_body(eat_ref, w1te_ref, b1_ref, out_ref):
    aT = eat_ref[...]
    y = jax.lax.dot_general(
        aT, w1te_ref[...], (((0,), (0,)), ((), ())),
        preferred_element_type=jnp.float32,
        precision=jax.lax.Precision.DEFAULT)
    h = jnp.maximum(y[:, :_D] + b1_ref[...], 0.0)
    w = 1.0 / (1.0 + jnp.exp(-y[:, _D:_D + 1]))
    out_ref[...] = h * w


def _payload(ea_t, W1, b1):
    e2 = jnp.zeros((4, _DW - _D), jnp.float32).at[2, 0].set(1.0)
    w1te = jnp.concatenate([W1.T, e2], axis=1)
    grid = (_EPAD // _EB,)
    return pl.pallas_call(
        _payload_body,
        grid=grid,
        in_specs=[
            pl.BlockSpec((4, _EB), lambda i: (0, i)),
            pl.BlockSpec((4, _DW), lambda i: (0, 0)),
            pl.BlockSpec((1, _D), lambda i: (0, 0)),
        ],
        out_specs=pl.BlockSpec((_EB, _D), lambda i: (i, 0)),
        out_shape=jax.ShapeDtypeStruct((_EPAD, _D), jnp.float32),
    )(ea_t, w1te, b1[None, :])


def _sigmoid16(v):
    return 1.0 / (1.0 + jnp.exp(-v))


def _sc_body(g_hbm, idx_hbm, ea2_hbm, zeros_hbm, outa_hbm, outs_hbm,
             rows0, rows1, idx0, idx1, ea0, ea1, s_v, acc_sh, sem0, sem1):
    c = lax.axis_index("c")
    s = lax.axis_index("s")
    wid = s * _NC + c
    r0 = s * _RPT
    rr = _NS * _RPT
    pltpu.sync_copy(zeros_hbm.at[pl.ds(r0, _RPT)], acc_sh.at[pl.ds(r0, _RPT)])

    @pl.when(s == _NS - 1)
    def _():
        pltpu.sync_copy(zeros_hbm.at[pl.ds(rr, _NACC - rr)],
                        acc_sh.at[pl.ds(rr, _NACC - rr)])

    def zbody(i, carry):
        s_v[pl.ds(i * _L, _L)] = jnp.zeros((_L,), jnp.float32)
        return carry

    lax.fori_loop(0, _NACC // _L, zbody, 0)

    ebase = wid * _EPT
    bufs = (rows0, rows1)
    idxs = (idx0, idx1)
    eas = (ea0, ea1)
    sems = (sem0, sem1)

    def start(off, slot):
        pltpu.async_copy(g_hbm.at[pl.ds(off, _CH)], bufs[slot], sems[slot])
        pltpu.async_copy(idx_hbm.at[pl.ds(off, _CH)], idxs[slot], sems[slot])
        pltpu.async_copy(ea2_hbm.at[pl.ds(off, _CH)], eas[slot], sems[slot])

    start(ebase, 0)

    def outer(gi, carry):
        for b in range(2):
            j = gi * 2 + b

            @pl.when(j + 1 < _CPT)
            def _():
                start(ebase + (j + 1) * _CH, 1 - b)

            pltpu.make_async_copy(
                g_hbm.at[pl.ds(0, _CH)], bufs[b], sems[b]).wait()
            pltpu.make_async_copy(
                idx_hbm.at[pl.ds(0, _CH)], idxs[b], sems[b]).wait()
            pltpu.make_async_copy(
                ea2_hbm.at[pl.ds(0, _CH)], eas[b], sems[b]).wait()
            pltpu.sync_copy(bufs[b], acc_sh.at[idxs[b]], add=True)
            for k in range(_CH // _L):
                idx16 = idxs[b][pl.ds(k * _L, _L)]
                wv = _sigmoid16(eas[b][pl.ds(k * _L, _L)])
                plsc.addupdate_scatter(s_v, [idx16], wv)
        return carry

    lax.fori_loop(0, _CPT // 2, outer, 0)

    plsc.subcore_barrier()
    pltpu.sync_copy(acc_sh.at[pl.ds(r0, _RPT)],
                    outa_hbm.at[c, pl.ds(r0, _RPT)])

    @pl.when(s == _NS - 1)
    def _():
        pltpu.sync_copy(acc_sh.at[pl.ds(rr, _N_NODES - rr)],
                        outa_hbm.at[c, pl.ds(rr, _N_NODES - rr)])

    pltpu.sync_copy(s_v, outs_hbm.at[wid])


def _sc_scatter(payload, idx1d, ea2_1d, zeros):
    mesh = plsc.VectorSubcoreMesh(core_axis_name="c", subcore_axis_name="s")
    fn = functools.partial(
        pl.kernel,
        mesh=mesh,
        compiler_params=pltpu.CompilerParams(needs_layout_passes=False),
        out_type=[
            jax.ShapeDtypeStruct((_NC, _N_NODES, _D), jnp.float32),
            jax.ShapeDtypeStruct((_NW, _NACC), jnp.float32),
        ],
        scratch_types=[
            pltpu.VMEM((_CH, _D), jnp.float32),
            pltpu.VMEM((_CH, _D), jnp.float32),
            pltpu.VMEM((_CH,), jnp.int32),
            pltpu.VMEM((_CH,), jnp.int32),
            pltpu.VMEM((_CH,), jnp.float32),
            pltpu.VMEM((_CH,), jnp.float32),
            pltpu.VMEM((_NACC,), jnp.float32),
            pltpu.VMEM_SHARED((_NACC, _D), jnp.float32),
            pltpu.SemaphoreType.DMA,
            pltpu.SemaphoreType.DMA,
        ],
    )(_sc_body)
    return fn(payload, idx1d, ea2_1d, zeros)


_NB = 1000


def _final_body(x_ref, acc_ref, s_ref, w2_ref, b2_ref, out_ref):
    a = acc_ref[0] + acc_ref[1]
    y = jax.lax.dot_general(
        a, w2_ref[...], (((1,), (1,)), ((), ())),
        preferred_element_type=jnp.float32,
        precision=jax.lax.Precision.HIGHEST)
    sw = jnp.sum(s_ref[...], axis=1, keepdims=True)
    out_ref[...] = x_ref[...] + y + sw * b2_ref[...]


def _final(x, acc, s_t, W2, b2):
    grid = (_N_NODES // _NB,)
    return pl.pallas_call(
        _final_body,
        grid=grid,
        in_specs=[
            pl.BlockSpec((_NB, _D), lambda i: (i, 0)),
            pl.BlockSpec((_NC, _NB, _D), lambda i: (0, i, 0)),
            pl.BlockSpec((_NB, _NW), lambda i: (i, 0)),
            pl.BlockSpec((_D, _D), lambda i: (0, 0)),
            pl.BlockSpec((1, _D), lambda i: (0, 0)),
        ],
        out_specs=pl.BlockSpec((_NB, _D), lambda i: (i, 0)),
        out_shape=jax.ShapeDtypeStruct((_N_NODES, _D), jnp.float32),
    )(x, acc, s_t, W2, b2[None, :])


def kernel(x, edge_index, edge_attr, W1, b1, W2, b2):
    npad = _EPAD - _N_EDGES
    dst = edge_index[1].astype(jnp.int32)
    pad_idx = _N_NODES + (jnp.arange(npad, dtype=jnp.int32) % 16)
    idx1d = jnp.concatenate([dst, pad_idx])
    ea_t = jnp.pad(edge_attr.T, ((0, 0), (0, npad)))
    ea2_1d = ea_t[2]
    payload = _payload(ea_t, W1, b1)
    zeros = jnp.zeros((_NACC, _D), jnp.float32)
    acc, s_parts = _sc_scatter(payload, idx1d, ea2_1d, zeros)
    return _final(x, acc, s_parts.T, W2, b2)


# speedup vs baseline: 5.0131x; 1.0976x over previous
import functools

import jax
import jax.numpy as jnp
from jax import lax
from jax.experimental import pallas as pl
from jax.experimental.pallas import tpu as pltpu
from jax.experimental.pallas import tpu_sc as plsc

_N_NODES = 10000
_N_EDGES = 320000
_D = 128

_NC, _NS = 2, 16
_NW = _NC * _NS
_CH = 128
_EPAD = 327680
_NHALF = 2
_HALF_E = _EPAD // _NHALF
_CPT = _HALF_E // (_NW * _CH)
_EPT = _CPT * _CH
_NACC = _N_NODES + 16
_RPT = 624
_L = 16


_EB = 10240


_DW = _D + 16


def _payload_body(eat_ref, w1te_ref, b1_ref, out_ref):
    aT = eat_ref[...]
    y = jax.lax.dot_general(
        aT, w1te_ref[...], (((0,), (0,)), ((), ())),
        preferred_element_type=jnp.float32,
        precision=jax.lax.Precision.DEFAULT)
    h = jnp.maximum(y[:, :_D] + b1_ref[...], 0.0)
    w = 1.0 / (1.0 + jnp.exp(-y[:, _D:_D + 1]))
    out_ref[...] = h * w


def _payload(ea_t, w1te, b1, half):
    grid = (_HALF_E // _EB,)
    blk_off = half * (_HALF_E // _EB)
    return pl.pallas_call(
        _payload_body,
        grid=grid,
        in_specs=[
            pl.BlockSpec((4, _EB), lambda i: (0, i + blk_off)),
            pl.BlockSpec((4, _DW), lambda i: (0, 0)),
            pl.BlockSpec((1, _D), lambda i: (0, 0)),
        ],
        out_specs=pl.BlockSpec((_EB, _D), lambda i: (i, 0)),
        out_shape=jax.ShapeDtypeStruct((_HALF_E, _D), jnp.float32),
    )(ea_t, w1te, b1[None, :])


def _sigmoid16(v):
    return 1.0 / (1.0 + jnp.exp(-v))


def _sc_body(g_hbm, idx_hbm, ea2_hbm, zeros_hbm, outa_hbm, outs_hbm,
             rows0, rows1, idx0, idx1, ea0, ea1, s_v, acc_sh, sem0, sem1):
    c = lax.axis_index("c")
    s = lax.axis_index("s")
    wid = s * _NC + c
    r0 = s * _RPT
    rr = _NS * _RPT
    pltpu.sync_copy(zeros_hbm.at[pl.ds(r0, _RPT)], acc_sh.at[pl.ds(r0, _RPT)])

    @pl.when(s == _NS - 1)
    def _():
        pltpu.sync_copy(zeros_hbm.at[pl.ds(rr, _NACC - rr)],
                        acc_sh.at[pl.ds(rr, _NACC - rr)])

    def zbody(i, carry):
        s_v[pl.ds(i * _L, _L)] = jnp.zeros((_L,), jnp.float32)
        return carry

    lax.fori_loop(0, _NACC // _L, zbody, 0)

    ebase = wid * _EPT
    bufs = (rows0, rows1)
    idxs = (idx0, idx1)
    eas = (ea0, ea1)
    sems = (sem0, sem1)

    def start(off, slot):
        pltpu.async_copy(g_hbm.at[pl.ds(off, _CH)], bufs[slot], sems[slot])
        pltpu.async_copy(idx_hbm.at[pl.ds(off, _CH)], idxs[slot], sems[slot])
        pltpu.async_copy(ea2_hbm.at[pl.ds(off, _CH)], eas[slot], sems[slot])

    start(ebase, 0)

    def outer(gi, carry):
        for b in range(2):
            j = gi * 2 + b

            @pl.when(j + 1 < _CPT)
            def _():
                start(ebase + (j + 1) * _CH, 1 - b)

            pltpu.make_async_copy(
                g_hbm.at[pl.ds(0, _CH)], bufs[b], sems[b]).wait()
            pltpu.make_async_copy(
                idx_hbm.at[pl.ds(0, _CH)], idxs[b], sems[b]).wait()
            pltpu.make_async_copy(
                ea2_hbm.at[pl.ds(0, _CH)], eas[b], sems[b]).wait()
            pltpu.sync_copy(bufs[b], acc_sh.at[idxs[b]], add=True)
            for k in range(_CH // _L):
                idx16 = idxs[b][pl.ds(k * _L, _L)]
                wv = _sigmoid16(eas[b][pl.ds(k * _L, _L)])
                plsc.addupdate_scatter(s_v, [idx16], wv)
        return carry

    lax.fori_loop(0, _CPT // 2, outer, 0)

    plsc.subcore_barrier()
    pltpu.sync_copy(acc_sh.at[pl.ds(r0, _RPT)],
                    outa_hbm.at[c, pl.ds(r0, _RPT)])

    @pl.when(s == _NS - 1)
    def _():
        pltpu.sync_copy(acc_sh.at[pl.ds(rr, _N_NODES - rr)],
                        outa_hbm.at[c, pl.ds(rr, _N_NODES - rr)])

    pltpu.sync_copy(s_v, outs_hbm.at[wid])


def _sc_scatter(payload, idx1d, ea2_1d, zeros):
    mesh = plsc.VectorSubcoreMesh(core_axis_name="c", subcore_axis_name="s")
    fn = functools.partial(
        pl.kernel,
        mesh=mesh,
        compiler_params=pltpu.CompilerParams(needs_layout_passes=False),
        out_type=[
            jax.ShapeDtypeStruct((_NC, _N_NODES, _D), jnp.float32),
            jax.ShapeDtypeStruct((_NW, _NACC), jnp.float32),
        ],
        scratch_types=[
            pltpu.VMEM((_CH, _D), jnp.float32),
            pltpu.VMEM((_CH, _D), jnp.float32),
            pltpu.VMEM((_CH,), jnp.int32),
            pltpu.VMEM((_CH,), jnp.int32),
            pltpu.VMEM((_CH,), jnp.float32),
            pltpu.VMEM((_CH,), jnp.float32),
            pltpu.VMEM((_NACC,), jnp.float32),
            pltpu.VMEM_SHARED((_NACC, _D), jnp.float32),
            pltpu.SemaphoreType.DMA,
            pltpu.SemaphoreType.DMA,
        ],
    )(_sc_body)
    return fn(payload, idx1d, ea2_1d, zeros)


_NB = 1000


def _final_body(x_ref, acca_ref, accb_ref, sa_ref, sb_ref,
                w2_ref, b2_ref, out_ref):
    a = (acca_ref[0] + acca_ref[1]) + (accb_ref[0] + accb_ref[1])
    y = jax.lax.dot_general(
        a, w2_ref[...], (((1,), (1,)), ((), ())),
        preferred_element_type=jnp.float32,
        precision=jax.lax.Precision.HIGHEST)
    sw = (jnp.sum(sa_ref[...], axis=1, keepdims=True)
          + jnp.sum(sb_ref[...], axis=1, keepdims=True))
    out_ref[...] = x_ref[...] + y + sw * b2_ref[...]


def _final(x, acc_a, acc_b, sa_t, sb_t, W2, b2):
    grid = (_N_NODES // _NB,)
    return pl.pallas_call(
        _final_body,
        grid=grid,
        in_specs=[
            pl.BlockSpec((_NB, _D), lambda i: (i, 0)),
            pl.BlockSpec((_NC, _NB, _D), lambda i: (0, i, 0)),
            pl.BlockSpec((_NC, _NB, _D), lambda i: (0, i, 0)),
            pl.BlockSpec((_NB, _NW), lambda i: (i, 0)),
            pl.BlockSpec((_NB, _NW), lambda i: (i, 0)),
            pl.BlockSpec((_D, _D), lambda i: (0, 0)),
            pl.BlockSpec((1, _D), lambda i: (0, 0)),
        ],
        out_specs=pl.BlockSpec((_NB, _D), lambda i: (i, 0)),
        out_shape=jax.ShapeDtypeStruct((_N_NODES, _D), jnp.float32),
    )(x, acc_a, acc_b, sa_t, sb_t, W2, b2[None, :])


def kernel(x, edge_index, edge_attr, W1, b1, W2, b2):
    npad = _EPAD - _N_EDGES
    dst = edge_index[1].astype(jnp.int32)
    pad_idx = _N_NODES + (jnp.arange(npad, dtype=jnp.int32) % 16)
    idx1d = jnp.concatenate([dst, pad_idx])
    ea_t = jnp.pad(edge_attr.T, ((0, 0), (0, npad)))
    ea2_1d = ea_t[2]
    e2 = jnp.zeros((4, _DW - _D), jnp.float32).at[2, 0].set(1.0)
    w1te = jnp.concatenate([W1.T, e2], axis=1)
    zeros = jnp.zeros((_NACC, _D), jnp.float32)

    pay_a = _payload(ea_t, w1te, b1, 0)
    pay_b = _payload(ea_t, w1te, b1, 1)
    acc_a, s_a = _sc_scatter(pay_a, idx1d[:_HALF_E], ea2_1d[:_HALF_E], zeros)
    acc_b, s_b = _sc_scatter(pay_b, idx1d[_HALF_E:], ea2_1d[_HALF_E:], zeros)
    return _final(x, acc_a, acc_b, s_a.T, s_b.T, W2, b2)
